# baseline (speedup 1.0000x reference)
import jax
import jax.numpy as jnp
from jax import lax
from jax.experimental import pallas as pl
from jax.experimental.pallas import tpu as pltpu
from jax.experimental.pallas import tpu_sc as plsc

N_NODES = 10000
N_EDGES = 320000
D_IN = 128
HIDDEN = 256
N_TYPES = 100

NC = 2
NS = 16
NW = NC * NS
NPAD = 10240
PAD_ROW = N_NODES
CHUNK = 128
CHS = 40
NSUP = 2
CH = CHS * NSUP
EPW = CH * CHUNK
EPAD = NW * EPW
ROWS_PER_SUB = NPAD // NS


def _deg_body(dst_hbm, out_hbm, dst_v, deg_v):
    c = lax.axis_index("c")
    s = lax.axis_index("s")
    wid = s * NC + c
    zeros = jnp.zeros((16,), jnp.float32)

    def zero_body(i, carry):
        deg_v[pl.ds(i * 16, 16)] = zeros
        return carry

    lax.fori_loop(0, NPAD // 16, zero_body, 0)
    pltpu.sync_copy(dst_hbm.at[wid], dst_v)
    ones = jnp.ones((16,), jnp.float32)

    def add_body(i, carry):
        idx = dst_v[pl.ds(i * 16, 16)]
        plsc.addupdate_scatter(deg_v, [idx], ones)
        return carry

    lax.fori_loop(0, EPW // 16, add_body, 0)
    pltpu.sync_copy(deg_v, out_hbm.at[wid])


_deg_call = pl.kernel(
    _deg_body,
    out_type=jax.ShapeDtypeStruct((NW, NPAD), jnp.float32),
    mesh=plsc.VectorSubcoreMesh(
        core_axis_name="c", subcore_axis_name="s", num_cores=NC, num_subcores=NS
    ),
    scratch_types=[
        pltpu.VMEM((EPW,), jnp.int32),
        pltpu.VMEM((NPAD,), jnp.float32),
    ],
    compiler_params=pltpu.CompilerParams(needs_layout_passes=False),
)


def _disy_body(degs_ref, x_ref, y_ref, dis_ref):
    deg = jnp.sum(degs_ref[...], axis=0) + 1.0
    dis = lax.rsqrt(deg)
    dis_ref[...] = dis[:, None]
    y_ref[pl.ds(0, N_NODES), :] = x_ref[...] * dis[:N_NODES, None]
    y_ref[pl.ds(N_NODES, NPAD - N_NODES), :] = jnp.zeros(
        (NPAD - N_NODES, D_IN), jnp.float32)


def _disy_call(degs, x_p):
    return pl.pallas_call(
        _disy_body,
        out_shape=[
            jax.ShapeDtypeStruct((NPAD, D_IN), jnp.float32),
            jax.ShapeDtypeStruct((NPAD, 1), jnp.float32),
        ],
    )(degs, x_p)


def _edge_body(y_hbm, src_hbm, dst_hbm, z_hbm, out_hbm,
               acc_sh, src_slab, dst_slab, rows0, rows1,
               semG0, semG1, semS0, semS1):
    c = lax.axis_index("c")
    s = lax.axis_index("s")
    wid = s * NC + c
    sub0 = s * ROWS_PER_SUB

    for r in range(ROWS_PER_SUB // CHUNK):
        pltpu.sync_copy(z_hbm, acc_sh.at[pl.ds(sub0 + r * CHUNK, CHUNK)])
    plsc.subcore_barrier()

    def super_body(sc, carry):
        pltpu.sync_copy(src_hbm.at[wid, pl.ds(sc * CHS, CHS)], src_slab)
        pltpu.sync_copy(dst_hbm.at[wid, pl.ds(sc * CHS, CHS)], dst_slab)
        pltpu.async_copy(y_hbm.at[src_slab.at[0]], rows0, semG0)

        def pair_body(g, carry2):
            j = 2 * g
            pltpu.async_copy(y_hbm.at[src_slab.at[j + 1]], rows1, semG1)
            pltpu.make_async_copy(y_hbm.at[src_slab.at[j]], rows0, semG0).wait()
            pltpu.sync_copy(rows0, acc_sh.at[dst_slab.at[j]], add=True)
            pltpu.async_copy(y_hbm.at[src_slab.at[j + 2]], rows0, semG0)
            pltpu.make_async_copy(
                y_hbm.at[src_slab.at[j + 1]], rows1, semG1).wait()
            pltpu.sync_copy(rows1, acc_sh.at[dst_slab.at[j + 1]], add=True)
            return carry2

        lax.fori_loop(0, CHS // 2 - 1, pair_body, 0)
        pltpu.async_copy(y_hbm.at[src_slab.at[CHS - 1]], rows1, semG1)
        pltpu.make_async_copy(y_hbm.at[src_slab.at[CHS - 2]], rows0, semG0).wait()
        pltpu.sync_copy(rows0, acc_sh.at[dst_slab.at[CHS - 2]], add=True)
        pltpu.make_async_copy(y_hbm.at[src_slab.at[CHS - 1]], rows1, semG1).wait()
        pltpu.sync_copy(rows1, acc_sh.at[dst_slab.at[CHS - 1]], add=True)
        return carry

    lax.fori_loop(0, NSUP, super_body, 0)
    plsc.subcore_barrier()
    pltpu.sync_copy(acc_sh.at[pl.ds(sub0, ROWS_PER_SUB)],
                    out_hbm.at[c, pl.ds(sub0, ROWS_PER_SUB)])


_edge_call = pl.kernel(
    _edge_body,
    out_type=jax.ShapeDtypeStruct((NC, NPAD, D_IN), jnp.float32),
    mesh=plsc.VectorSubcoreMesh(
        core_axis_name="c", subcore_axis_name="s", num_cores=NC, num_subcores=NS
    ),
    scratch_types=[
        pltpu.VMEM_SHARED((NPAD, D_IN), jnp.float32),
        pltpu.VMEM((CHS, CHUNK), jnp.int32),
        pltpu.VMEM((CHS, CHUNK), jnp.int32),
        pltpu.VMEM((CHUNK, D_IN), jnp.float32),
        pltpu.VMEM((CHUNK, D_IN), jnp.float32),
        pltpu.SemaphoreType.DMA,
        pltpu.SemaphoreType.DMA,
        pltpu.SemaphoreType.DMA,
        pltpu.SemaphoreType.DMA,
    ],
    compiler_params=pltpu.CompilerParams(needs_layout_passes=False),
)


def _final_body(acc_ref, y_ref, dis_ref, ct_ref, w_ref, b_ref, out_ref):
    agg = acc_ref[0] + acc_ref[1] + y_ref[...]
    outv = agg * dis_ref[...]
    h = jnp.dot(outv, w_ref[...], preferred_element_type=jnp.float32) + b_ref[...]
    h = jnp.maximum(h, 0.0)
    tids = lax.broadcasted_iota(jnp.int32, (NPAD, 128), 1)
    onehot = (ct_ref[...] == tids).astype(jnp.float32)
    sums = jnp.dot(onehot.T, h, preferred_element_type=jnp.float32)
    counts = jnp.sum(onehot, axis=0)
    out_ref[...] = sums / jnp.maximum(counts, 1.0)[:, None]


def _final_call(accs, y, dis, ct_p, W_cell, b2):
    return pl.pallas_call(
        _final_body,
        out_shape=jax.ShapeDtypeStruct((128, HIDDEN), jnp.float32),
    )(accs, y, dis, ct_p, W_cell, b2)


def kernel(x, edge_index, cell_type_batch, W_cell, b_cell):
    pad_e = EPAD - N_EDGES
    src = edge_index[0]
    dst = edge_index[1]
    pad_idx = PAD_ROW + (jnp.arange(pad_e, dtype=jnp.int32) % (NPAD - N_NODES))
    src_p = jnp.concatenate([src, pad_idx]).reshape(NW, CH, CHUNK)
    dst_p = jnp.concatenate([dst, pad_idx]).reshape(NW, CH, CHUNK)
    dst_flat = dst_p.reshape(NW, EPW)
    ct_p = jnp.concatenate(
        [cell_type_batch,
         jnp.full((NPAD - N_NODES,), 127, jnp.int32)])[:, None]
    zeros_hbm = jnp.zeros((CHUNK, D_IN), jnp.float32)

    degs = _deg_call(dst_flat)
    y, dis = _disy_call(degs, x)
    accs = _edge_call(y, src_p, dst_p, zeros_hbm)
    pooled = _final_call(accs, y, dis, ct_p, W_cell, b_cell[None, :])
    return pooled[:N_TYPES]


# speedup vs baseline: 1.0416x; 1.0416x over previous
import jax
import jax.numpy as jnp
from jax import lax
from jax.experimental import pallas as pl
from jax.experimental.pallas import tpu as pltpu
from jax.experimental.pallas import tpu_sc as plsc

N_NODES = 10000
N_EDGES = 320000
D_IN = 128
HIDDEN = 256
N_TYPES = 100

NC = 2
NS = 16
NW = NC * NS
NPAD = 10240
PAD_ROW = N_NODES
CHUNK = 128
CHS = 40
NSUP = 2
CH = CHS * NSUP
EPW = CH * CHUNK
EPAD = NW * EPW
ROWS_PER_SUB = NPAD // NS


def _deg_body(dst_hbm, out_hbm, dst_v, deg_v):
    c = lax.axis_index("c")
    s = lax.axis_index("s")
    wid = s * NC + c
    zeros = jnp.zeros((16,), jnp.float32)

    def zero_body(i, carry):
        deg_v[pl.ds(i * 16, 16)] = zeros
        return carry

    lax.fori_loop(0, NPAD // 16, zero_body, 0)
    pltpu.sync_copy(dst_hbm.at[wid], dst_v)
    ones = jnp.ones((16,), jnp.float32)

    def add_body(i, carry):
        idx = dst_v[pl.ds(i * 16, 16)]
        plsc.addupdate_scatter(deg_v, [idx], ones)
        return carry

    lax.fori_loop(0, EPW // 16, add_body, 0)
    pltpu.sync_copy(deg_v, out_hbm.at[wid])


_deg_call = pl.kernel(
    _deg_body,
    out_type=jax.ShapeDtypeStruct((NW, NPAD), jnp.float32),
    mesh=plsc.VectorSubcoreMesh(
        core_axis_name="c", subcore_axis_name="s", num_cores=NC, num_subcores=NS
    ),
    scratch_types=[
        pltpu.VMEM((EPW,), jnp.int32),
        pltpu.VMEM((NPAD,), jnp.float32),
    ],
    compiler_params=pltpu.CompilerParams(needs_layout_passes=False),
)


def _disy_body(degs_ref, x_ref, y_ref, dis_ref):
    deg = jnp.sum(degs_ref[...], axis=0) + 1.0
    dis = lax.rsqrt(deg)
    dis_ref[...] = dis[:, None]
    y_ref[pl.ds(0, N_NODES), :] = x_ref[...] * dis[:N_NODES, None]
    y_ref[pl.ds(N_NODES, NPAD - N_NODES), :] = jnp.zeros(
        (NPAD - N_NODES, D_IN), jnp.float32)


def _disy_call(degs, x_p):
    return pl.pallas_call(
        _disy_body,
        out_shape=[
            jax.ShapeDtypeStruct((NPAD, D_IN), jnp.float32),
            jax.ShapeDtypeStruct((NPAD, 1), jnp.float32),
        ],
    )(degs, x_p)


def _edge_body(y_hbm, src_hbm, dst_hbm, z_hbm, out_hbm,
               acc_sh, src_slab, dst_slab, rows0, rows1,
               semG0, semG1, semS0, semS1):
    c = lax.axis_index("c")
    s = lax.axis_index("s")
    wid = s * NC + c
    sub0 = s * ROWS_PER_SUB

    pltpu.sync_copy(z_hbm.at[pl.ds(sub0, ROWS_PER_SUB)],
                    acc_sh.at[pl.ds(sub0, ROWS_PER_SUB)])
    plsc.subcore_barrier()

    def super_body(sc, carry):
        pltpu.sync_copy(src_hbm.at[wid, pl.ds(sc * CHS, CHS)], src_slab)
        pltpu.sync_copy(dst_hbm.at[wid, pl.ds(sc * CHS, CHS)], dst_slab)
        pltpu.async_copy(y_hbm.at[src_slab.at[0]], rows0, semG0)

        def pair_body(g, carry2):
            j = 2 * g
            pltpu.async_copy(y_hbm.at[src_slab.at[j + 1]], rows1, semG1)
            pltpu.make_async_copy(y_hbm.at[src_slab.at[j]], rows0, semG0).wait()
            pltpu.sync_copy(rows0, acc_sh.at[dst_slab.at[j]], add=True)
            pltpu.async_copy(y_hbm.at[src_slab.at[j + 2]], rows0, semG0)
            pltpu.make_async_copy(
                y_hbm.at[src_slab.at[j + 1]], rows1, semG1).wait()
            pltpu.sync_copy(rows1, acc_sh.at[dst_slab.at[j + 1]], add=True)
            return carry2

        lax.fori_loop(0, CHS // 2 - 1, pair_body, 0)
        pltpu.async_copy(y_hbm.at[src_slab.at[CHS - 1]], rows1, semG1)
        pltpu.make_async_copy(y_hbm.at[src_slab.at[CHS - 2]], rows0, semG0).wait()
        pltpu.sync_copy(rows0, acc_sh.at[dst_slab.at[CHS - 2]], add=True)
        pltpu.make_async_copy(y_hbm.at[src_slab.at[CHS - 1]], rows1, semG1).wait()
        pltpu.sync_copy(rows1, acc_sh.at[dst_slab.at[CHS - 1]], add=True)
        return carry

    lax.fori_loop(0, NSUP, super_body, 0)
    plsc.subcore_barrier()
    pltpu.sync_copy(acc_sh.at[pl.ds(sub0, ROWS_PER_SUB)],
                    out_hbm.at[c, pl.ds(sub0, ROWS_PER_SUB)])


_edge_call = pl.kernel(
    _edge_body,
    out_type=jax.ShapeDtypeStruct((NC, NPAD, D_IN), jnp.float32),
    mesh=plsc.VectorSubcoreMesh(
        core_axis_name="c", subcore_axis_name="s", num_cores=NC, num_subcores=NS
    ),
    scratch_types=[
        pltpu.VMEM_SHARED((NPAD, D_IN), jnp.float32),
        pltpu.VMEM((CHS, CHUNK), jnp.int32),
        pltpu.VMEM((CHS, CHUNK), jnp.int32),
        pltpu.VMEM((CHUNK, D_IN), jnp.float32),
        pltpu.VMEM((CHUNK, D_IN), jnp.float32),
        pltpu.SemaphoreType.DMA,
        pltpu.SemaphoreType.DMA,
        pltpu.SemaphoreType.DMA,
        pltpu.SemaphoreType.DMA,
    ],
    compiler_params=pltpu.CompilerParams(needs_layout_passes=False),
)


def _final_body(acc_ref, y_ref, dis_ref, ct_ref, w_ref, b_ref, out_ref):
    agg = acc_ref[0] + acc_ref[1] + y_ref[...]
    outv = agg * dis_ref[...]
    h = jnp.dot(outv, w_ref[...], preferred_element_type=jnp.float32) + b_ref[...]
    h = jnp.maximum(h, 0.0)
    tids = lax.broadcasted_iota(jnp.int32, (NPAD, 128), 1)
    onehot = (ct_ref[...] == tids).astype(jnp.float32)
    sums = jnp.dot(onehot.T, h, preferred_element_type=jnp.float32)
    counts = jnp.sum(onehot, axis=0)
    out_ref[...] = sums / jnp.maximum(counts, 1.0)[:, None]


def _final_call(accs, y, dis, ct_p, W_cell, b2):
    return pl.pallas_call(
        _final_body,
        out_shape=jax.ShapeDtypeStruct((128, HIDDEN), jnp.float32),
    )(accs, y, dis, ct_p, W_cell, b2)


def kernel(x, edge_index, cell_type_batch, W_cell, b_cell):
    pad_e = EPAD - N_EDGES
    src = edge_index[0]
    dst = edge_index[1]
    pad_idx = PAD_ROW + (jnp.arange(pad_e, dtype=jnp.int32) % (NPAD - N_NODES))
    src_p = jnp.concatenate([src, pad_idx]).reshape(NW, CH, CHUNK)
    dst_p = jnp.concatenate([dst, pad_idx]).reshape(NW, CH, CHUNK)
    dst_flat = dst_p.reshape(NW, EPW)
    ct_p = jnp.concatenate(
        [cell_type_batch,
         jnp.full((NPAD - N_NODES,), 127, jnp.int32)])[:, None]
    zeros_hbm = jnp.zeros((NPAD, D_IN), jnp.float32)

    degs = _deg_call(dst_flat)
    y, dis = _disy_call(degs, x)
    accs = _edge_call(y, src_p, dst_p, zeros_hbm)
    pooled = _final_call(accs, y, dis, ct_p, W_cell, b_cell[None, :])
    return pooled[:N_TYPES]
